# baseline (speedup 1.0000x reference)
import functools

import jax
import jax.numpy as jnp
from jax import lax
from jax.experimental import pallas as pl
from jax.experimental.pallas import tpu as pltpu
from jax.experimental.pallas import tpu_sc as plsc

EMBED_DIM = 64
BATCH = 16384
NUM_SMALL = 7
_TC_GRID = 2
_WINDOW = 1024

_info = plsc.get_sparse_core_info()
_NC, _NS = _info.num_cores, _info.num_subcores
_NW = _NC * _NS
_BPW = BATCH // _NW


def _sc_body(*refs):
    tables = refs[0:NUM_SMALL]
    idxs = refs[NUM_SMALL:2 * NUM_SMALL]
    outs = refs[2 * NUM_SMALL:3 * NUM_SMALL]
    idx_v, row_v, sem = refs[3 * NUM_SMALL:]

    wid = lax.axis_index("s") * _NC + lax.axis_index("c")
    base = wid * _BPW
    for t in range(NUM_SMALL):
        pltpu.sync_copy(idxs[t].at[pl.ds(base, _BPW)], idx_v)

        def fire(g, carry, t=t):
            vec = idx_v[pl.ds(g * 16, 16)]
            for j in range(16):
                pltpu.async_copy(tables[t].at[vec[j]],
                                 row_v.at[g * 16 + j], sem)
            return carry
        lax.fori_loop(0, _BPW // 16, fire, 0)

        def drain(i, carry, t=t):
            pltpu.make_async_copy(tables[t].at[0], row_v.at[i], sem).wait()
            return carry
        lax.fori_loop(0, _BPW, drain, 0)
        pltpu.sync_copy(row_v, outs[t].at[pl.ds(base, _BPW)])


def _visit_body(idx_ref, table_ref, out_ref, sem):
    half = BATCH // _TC_GRID
    base = pl.program_id(0) * half

    def fire(i, carry):
        r = idx_ref[base + i]
        pltpu.make_async_copy(table_ref.at[pl.ds(r, 1)],
                              out_ref.at[pl.ds(base + i, 1)], sem).start()

        @pl.when(i >= _WINDOW)
        def _():
            pltpu.make_async_copy(table_ref.at[pl.ds(0, 1)],
                                  out_ref.at[pl.ds(base, 1)], sem).wait()
        return carry
    lax.fori_loop(0, half, fire, 0)

    def drain(i, carry):
        pltpu.make_async_copy(table_ref.at[pl.ds(0, 1)],
                              out_ref.at[pl.ds(base, 1)], sem).wait()
        return carry
    lax.fori_loop(0, _WINDOW, drain, 0)


@jax.jit
def kernel(patient_emb, visit_emb, symptom_emb, procedure_emb, disease_emb,
           drug_emb, anatomy_emb, pharmaclass_emb,
           patient_node_id, visit_node_id, symptom_node_id, procedure_node_id,
           disease_node_id, drug_node_id, anatomy_node_id, pharmaclass_node_id):
    out_type = tuple(
        jax.ShapeDtypeStruct((BATCH, EMBED_DIM), jnp.float32)
        for _ in range(NUM_SMALL)
    )
    sc = functools.partial(
        pl.kernel,
        mesh=plsc.VectorSubcoreMesh(core_axis_name="c", subcore_axis_name="s"),
        out_type=out_type,
        scratch_types=[
            pltpu.VMEM((_BPW,), jnp.int32),
            pltpu.VMEM((_BPW, EMBED_DIM), jnp.float32),
            pltpu.SemaphoreType.DMA,
        ],
        compiler_params=pltpu.CompilerParams(needs_layout_passes=False),
    )(_sc_body)
    x_p, x_s, x_pr, x_di, x_dr, x_a, x_ph = sc(
        patient_emb, symptom_emb, procedure_emb, disease_emb,
        drug_emb, anatomy_emb, pharmaclass_emb,
        patient_node_id, symptom_node_id, procedure_node_id,
        disease_node_id, drug_node_id, anatomy_node_id, pharmaclass_node_id)

    x_visit = pl.pallas_call(
        _visit_body,
        grid_spec=pltpu.PrefetchScalarGridSpec(
            num_scalar_prefetch=1,
            grid=(_TC_GRID,),
            in_specs=[pl.BlockSpec(memory_space=pltpu.MemorySpace.HBM)],
            out_specs=pl.BlockSpec(memory_space=pltpu.MemorySpace.HBM),
            scratch_shapes=[pltpu.SemaphoreType.DMA],
        ),
        out_shape=jax.ShapeDtypeStruct((BATCH, EMBED_DIM), jnp.float32),
        compiler_params=pltpu.CompilerParams(
            dimension_semantics=("parallel",)),
    )(visit_node_id, visit_emb)

    return (x_p, x_visit, x_s, x_pr, x_di, x_dr, x_a, x_ph)


# speedup vs baseline: 1.4984x; 1.4984x over previous
import functools

import jax
import jax.numpy as jnp
from jax import lax
from jax.experimental import pallas as pl
from jax.experimental.pallas import tpu as pltpu
from jax.experimental.pallas import tpu_sc as plsc

EMBED_DIM = 64
BATCH = 16384
NUM_SMALL = 7

_info = plsc.get_sparse_core_info()
_NC, _NS = _info.num_cores, _info.num_subcores
_NW = _NC * _NS
_BPW = BATCH // _NW


def _gather_body(num_tables, *refs):
    tables = refs[0:num_tables]
    idxs = refs[num_tables:2 * num_tables]
    outs = refs[2 * num_tables:3 * num_tables]
    idx_v, row_v, sem = refs[3 * num_tables:]

    wid = lax.axis_index("s") * _NC + lax.axis_index("c")
    base = wid * _BPW
    for t in range(num_tables):
        pltpu.sync_copy(idxs[t].at[pl.ds(base, _BPW)], idx_v)

        def fire(g, carry, t=t):
            vec = idx_v[pl.ds(g * 16, 16)]
            for j in range(16):
                pltpu.async_copy(tables[t].at[vec[j]],
                                 row_v.at[g * 16 + j], sem)
            return carry
        lax.fori_loop(0, _BPW // 16, fire, 0)

        def drain(i, carry, t=t):
            pltpu.make_async_copy(tables[t].at[0], row_v.at[i], sem).wait()
            return carry
        lax.fori_loop(0, _BPW, drain, 0)
        pltpu.sync_copy(row_v, outs[t].at[pl.ds(base, _BPW)])


def _make_gather(num_tables):
    out_type = tuple(
        jax.ShapeDtypeStruct((BATCH, EMBED_DIM), jnp.float32)
        for _ in range(num_tables)
    )
    return functools.partial(
        pl.kernel,
        mesh=plsc.VectorSubcoreMesh(core_axis_name="c", subcore_axis_name="s"),
        out_type=out_type,
        scratch_types=[
            pltpu.VMEM((_BPW,), jnp.int32),
            pltpu.VMEM((_BPW, EMBED_DIM), jnp.float32),
            pltpu.SemaphoreType.DMA,
        ],
        compiler_params=pltpu.CompilerParams(needs_layout_passes=False),
    )(functools.partial(_gather_body, num_tables))


@jax.jit
def kernel(patient_emb, visit_emb, symptom_emb, procedure_emb, disease_emb,
           drug_emb, anatomy_emb, pharmaclass_emb,
           patient_node_id, visit_node_id, symptom_node_id, procedure_node_id,
           disease_node_id, drug_node_id, anatomy_node_id, pharmaclass_node_id):
    x_p, x_s, x_pr, x_di, x_dr, x_a, x_ph = _make_gather(NUM_SMALL)(
        patient_emb, symptom_emb, procedure_emb, disease_emb,
        drug_emb, anatomy_emb, pharmaclass_emb,
        patient_node_id, symptom_node_id, procedure_node_id,
        disease_node_id, drug_node_id, anatomy_node_id, pharmaclass_node_id)
    (x_visit,) = _make_gather(1)(visit_emb, visit_node_id)
    return (x_p, x_visit, x_s, x_pr, x_di, x_dr, x_a, x_ph)


# speedup vs baseline: 1.5107x; 1.0082x over previous
import functools

import jax
import jax.numpy as jnp
from jax import lax
from jax.experimental import pallas as pl
from jax.experimental.pallas import tpu as pltpu
from jax.experimental.pallas import tpu_sc as plsc

EMBED_DIM = 64
BATCH = 16384
NUM_SMALL = 7

_info = plsc.get_sparse_core_info()
_NC, _NS = _info.num_cores, _info.num_subcores
_NW = _NC * _NS
_BPW = BATCH // _NW


def _gather_body(num_tables, *refs):
    tables = refs[0:num_tables]
    idxs = refs[num_tables:2 * num_tables]
    outs = refs[2 * num_tables:3 * num_tables]
    idx_v, row_v, sem = refs[3 * num_tables:]

    wid = lax.axis_index("s") * _NC + lax.axis_index("c")
    base = wid * _BPW
    for t in range(num_tables):
        pltpu.sync_copy(idxs[t].at[pl.ds(base, _BPW)], idx_v)

        def fire(g, carry, t=t):
            vec = idx_v[pl.ds(g * 16, 16)]
            for j in range(16):
                pltpu.async_copy(tables[t].at[vec[j]],
                                 row_v.at[g * 16 + j], sem)
            return carry
        lax.fori_loop(0, _BPW // 16, fire, 0)

        def drain(i, carry, t=t):
            pltpu.make_async_copy(tables[t].at[0], row_v.at[i], sem).wait()
            return carry
        lax.fori_loop(0, _BPW, drain, 0)
        pltpu.sync_copy(row_v, outs[t].at[pl.ds(base, _BPW)])


def _make_gather(num_tables):
    out_type = tuple(
        jax.ShapeDtypeStruct((BATCH, EMBED_DIM), jnp.float32)
        for _ in range(num_tables)
    )
    return functools.partial(
        pl.kernel,
        mesh=plsc.VectorSubcoreMesh(core_axis_name="c", subcore_axis_name="s"),
        out_type=out_type,
        scratch_types=[
            pltpu.VMEM((_BPW,), jnp.int32),
            pltpu.VMEM((_BPW, EMBED_DIM), jnp.float32),
            pltpu.SemaphoreType.DMA,
        ],
        compiler_params=pltpu.CompilerParams(needs_layout_passes=False),
    )(functools.partial(_gather_body, num_tables))


@jax.jit
def kernel(patient_emb, visit_emb, symptom_emb, procedure_emb, disease_emb,
           drug_emb, anatomy_emb, pharmaclass_emb,
           patient_node_id, visit_node_id, symptom_node_id, procedure_node_id,
           disease_node_id, drug_node_id, anatomy_node_id, pharmaclass_node_id):
    g = _make_gather(1)
    (x_p,) = g(patient_emb, patient_node_id)
    (x_visit,) = g(visit_emb, visit_node_id)
    (x_s,) = g(symptom_emb, symptom_node_id)
    (x_pr,) = g(procedure_emb, procedure_node_id)
    (x_di,) = g(disease_emb, disease_node_id)
    (x_dr,) = g(drug_emb, drug_node_id)
    (x_a,) = g(anatomy_emb, anatomy_node_id)
    (x_ph,) = g(pharmaclass_emb, pharmaclass_node_id)
    return (x_p, x_visit, x_s, x_pr, x_di, x_dr, x_a, x_ph)
